# baseline (speedup 1.0000x reference)
import numpy as np

import jax
import jax.numpy as jnp
from jax.experimental import pallas as pl
from jax.experimental.pallas import tpu as pltpu
from jax.experimental.shard_map import shard_map
from jax.sharding import Mesh, PartitionSpec as P

H3 = 384
FEAT = 256
T = 8
BLOCK_N = 1000
NCHAINS = 5


def _sigmoid(x):
    return 0.5 * jnp.tanh(0.5 * x) + 0.5


def _gru_step(gi, h, whh, brz, b_in, b_hn):
    if h is None:
        gh_rz = brz
        gh_n = b_hn
    else:
        gh = jnp.dot(h.astype(jnp.bfloat16), whh,
                     preferred_element_type=jnp.float32)
        gh_rz = gh[:, :2 * H3] + brz
        gh_n = gh[:, 2 * H3:] + b_hn
    rz = _sigmoid(gi[:, :2 * H3] + gh_rz)
    r = rz[:, :H3]
    z = rz[:, H3:]
    n = jnp.tanh(gi[:, 2 * H3:] + b_in + r * gh_n)
    if h is None:
        return n - z * n
    return n + z * (h - n)


def _gru_body(x1_ref, x4_ref, wih_ref, whh_ref, brz_ref, bin_ref, bhn_ref,
              d1_ref, d4_ref, dc_ref, out_ref):
    whh = whh_ref[...]
    brz = brz_ref[...]
    b_in = bin_ref[...]
    b_hn = bhn_ref[...]

    xall = x1_ref[...].reshape(T * BLOCK_N, FEAT)
    gi_all = jnp.dot(xall, wih_ref[...],
                     preferred_element_type=jnp.float32
                     ).reshape(T, BLOCK_N, 3 * H3)

    hb = BLOCK_N // NCHAINS
    hs = [None] * NCHAINS
    for t in range(T):
        gi = gi_all[t]
        for k in range(NCHAINS):
            hs[k] = _gru_step(gi[k * hb:(k + 1) * hb], hs[k],
                              whh, brz, b_in, b_hn)

    h = jnp.concatenate(hs, axis=0)
    delta = jnp.sum(h * d1_ref[...], axis=1, keepdims=True)
    delta = delta + jnp.sum(x4_ref[...] * d4_ref[...], axis=1, keepdims=True)
    delta = delta + dc_ref[0, 0]
    p0 = _sigmoid(delta)
    out_ref[:, 0:1] = p0
    out_ref[:, 1:2] = 1.0 - p0


def _run_shard(x1, x4, wih_t, whh_t, brz, b_in, b_hn, d1, d4, dc):
    x1t = jnp.swapaxes(x1, 0, 1).astype(jnp.bfloat16)
    n_local = x1t.shape[1]
    ccl = x4.shape[1]
    grid = (n_local // BLOCK_N,)
    out = pl.pallas_call(
        _gru_body,
        grid=grid,
        in_specs=[
            pl.BlockSpec((T, BLOCK_N, FEAT), lambda i: (0, i, 0)),
            pl.BlockSpec((BLOCK_N, ccl), lambda i: (i, 0)),
            pl.BlockSpec((FEAT, 3 * H3), lambda i: (0, 0)),
            pl.BlockSpec((H3, 3 * H3), lambda i: (0, 0)),
            pl.BlockSpec((1, 2 * H3), lambda i: (0, 0)),
            pl.BlockSpec((1, H3), lambda i: (0, 0)),
            pl.BlockSpec((1, H3), lambda i: (0, 0)),
            pl.BlockSpec((1, H3), lambda i: (0, 0)),
            pl.BlockSpec((1, ccl), lambda i: (0, 0)),
            pl.BlockSpec((1, 1), lambda i: (0, 0)),
        ],
        out_specs=pl.BlockSpec((BLOCK_N, 2), lambda i: (i, 0)),
        out_shape=jax.ShapeDtypeStruct((n_local, 2), jnp.float32),
        compiler_params=pltpu.CompilerParams(
            dimension_semantics=("arbitrary",)),
    )(x1t, x4, wih_t, whh_t, brz, b_in, b_hn, d1, d4, dc)
    return out.T


def kernel(x1, x4, edge_index, W_ih, W_hh, b_ih, b_hh,
           W1, b1, W2, b2, W3, b3, W4, b4, W7, b7):
    n = x1.shape[0]

    w7 = W7[0] - W7[1]
    d1 = (W1.T @ w7[:128])[None, :]
    d4 = (W4.T @ w7[384:])[None, :]
    dc = (b1 @ w7[:128] + b2 @ w7[128:256] + b3 @ w7[256:384]
          + b4 @ w7[384:] + (b7[0] - b7[1])).reshape(1, 1)

    wih_t = W_ih.T.astype(jnp.bfloat16)
    whh_t = W_hh.T.astype(jnp.bfloat16)
    brz = (b_ih[:2 * H3] + b_hh[:2 * H3])[None, :]
    b_in = b_ih[None, 2 * H3:]
    b_hn = b_hh[None, 2 * H3:]

    devs = jax.devices()
    ndev = len(devs) if n % (len(devs) * BLOCK_N) == 0 else 1
    if ndev == 1:
        return _run_shard(x1, x4, wih_t, whh_t, brz, b_in, b_hn, d1, d4, dc)

    mesh = Mesh(np.array(devs[:ndev]), ("d",))
    rep = P(None, None)
    fn = shard_map(
        _run_shard, mesh=mesh,
        in_specs=(P("d", None, None), P("d", None),
                  rep, rep, rep, rep, rep, rep, rep, rep),
        out_specs=P(None, "d"), check_rep=False,
    )
    return fn(x1, x4, wih_t, whh_t, brz, b_in, b_hn, d1, d4, dc)


# speedup vs baseline: 3.1049x; 3.1049x over previous
import jax
import jax.numpy as jnp
from jax.experimental import pallas as pl
from jax.experimental.pallas import tpu as pltpu

H3 = 384
FEAT = 256
T = 8
BLOCK_N = 2000
BOUNDS = (0, 400, 800, 1200, 1600, 2000)


def _sigmoid(x):
    return 0.5 * jnp.tanh(0.5 * x) + 0.5


def _gru_body(x1_ref, x4_ref, wih_ref, wrz_ref, whhn_ref, brz_ref, bin_ref,
              bhn_ref, d1_ref, d4_ref, dc_ref, out_ref):
    wrz = wrz_ref[...]
    whhn = whhn_ref[...]
    brz = brz_ref[...]
    b_in = bin_ref[...]
    b_hn = bhn_ref[...]

    xall = x1_ref[...]

    g0 = jnp.dot(xall[0], wih_ref[...],
                 preferred_element_type=jnp.float32)
    rz0 = _sigmoid(g0[:, :2 * H3] + brz)
    n0 = jnp.tanh(g0[:, 2 * H3:] + b_in + rz0[:, :H3] * b_hn)
    h0 = n0 - rz0[:, H3:] * n0

    gin_all = jnp.dot(xall[1:].reshape((T - 1) * BLOCK_N, FEAT),
                      wih_ref[:, 2 * H3:],
                      preferred_element_type=jnp.float32
                      ).reshape(T - 1, BLOCK_N, H3)

    hs = [h0[b0:b1] for b0, b1 in zip(BOUNDS[:-1], BOUNDS[1:])]
    for t in range(1, T):
        for k in range(len(hs)):
            b0, b1 = BOUNDS[k], BOUNDS[k + 1]
            hbf = hs[k].astype(jnp.bfloat16)
            xh = jnp.concatenate([xall[t, b0:b1], hbf], axis=1)
            grz = jnp.dot(xh, wrz, preferred_element_type=jnp.float32)
            ghn = jnp.dot(hbf, whhn, preferred_element_type=jnp.float32)
            rz = _sigmoid(grz + brz)
            r = rz[:, :H3]
            z = rz[:, H3:]
            n = jnp.tanh(gin_all[t - 1, b0:b1] + b_in + r * (ghn + b_hn))
            hs[k] = n + z * (hs[k] - n)

    h = jnp.concatenate(hs, axis=0)
    delta = jnp.sum(h * d1_ref[...], axis=1, keepdims=True)
    delta = delta + jnp.sum(x4_ref[...] * d4_ref[...], axis=1, keepdims=True)
    delta = delta + dc_ref[0, 0]
    p0 = _sigmoid(delta)
    out_ref[:, 0:1] = p0
    out_ref[:, 1:2] = 1.0 - p0


def kernel(x1, x4, edge_index, W_ih, W_hh, b_ih, b_hh,
           W1, b1, W2, b2, W3, b3, W4, b4, W7, b7):
    n = x1.shape[0]
    ccl = x4.shape[1]

    w7 = W7[0] - W7[1]
    d1 = W1.T @ w7[:128]
    d4 = W4.T @ w7[384:]
    dc = (b1 @ w7[:128] + b2 @ w7[128:256] + b3 @ w7[256:384]
          + b4 @ w7[384:] + (b7[0] - b7[1]))

    wih_t = W_ih.T.astype(jnp.bfloat16)
    whh_t = W_hh.T.astype(jnp.bfloat16)
    wrz = jnp.concatenate([wih_t[:, :2 * H3], whh_t[:, :2 * H3]], axis=0)
    whhn = whh_t[:, 2 * H3:]

    grid = (n // BLOCK_N,)
    out = pl.pallas_call(
        _gru_body,
        grid=grid,
        in_specs=[
            pl.BlockSpec((T, BLOCK_N, FEAT), lambda i: (0, i, 0)),
            pl.BlockSpec((BLOCK_N, ccl), lambda i: (i, 0)),
            pl.BlockSpec((FEAT, 3 * H3), lambda i: (0, 0)),
            pl.BlockSpec((FEAT + H3, 2 * H3), lambda i: (0, 0)),
            pl.BlockSpec((H3, H3), lambda i: (0, 0)),
            pl.BlockSpec((1, 2 * H3), lambda i: (0, 0)),
            pl.BlockSpec((1, H3), lambda i: (0, 0)),
            pl.BlockSpec((1, H3), lambda i: (0, 0)),
            pl.BlockSpec((1, H3), lambda i: (0, 0)),
            pl.BlockSpec((1, ccl), lambda i: (0, 0)),
            pl.BlockSpec((1, 1), lambda i: (0, 0)),
        ],
        out_specs=pl.BlockSpec((BLOCK_N, 2), lambda i: (i, 0)),
        out_shape=jax.ShapeDtypeStruct((n, 2), jnp.float32),
        compiler_params=pltpu.CompilerParams(
            dimension_semantics=("parallel",)),
    )(
        jnp.swapaxes(x1, 0, 1).astype(jnp.bfloat16),
        x4,
        wih_t,
        wrz,
        whhn,
        (b_ih[:2 * H3] + b_hh[:2 * H3])[None, :],
        b_ih[None, 2 * H3:],
        b_hh[None, 2 * H3:],
        d1[None, :],
        d4[None, :],
        dc.reshape(1, 1),
    )
    return out.T
